# baseline (speedup 1.0000x reference)
import functools

import jax
import jax.numpy as jnp
from jax import lax
from jax.experimental import pallas as pl
from jax.experimental.pallas import tpu as pltpu
from jax.experimental.pallas import tpu_sc as plsc

_HIDDEN = 4096
_EXPERTS = 64
_TM = 1024

_NC, _NS, _L = 2, 16, 16
_NW = _NC * _NS


def _matmul_tile(x_ref, w_ref, lt_ref):
    logits = jax.lax.dot_general(
        x_ref[...], w_ref[...], (((1,), (1,)), ((), ())),
        preferred_element_type=jnp.float32)
    lt_ref[...] = logits.T


def _tc_logits_t(x, W):
    n_tokens = x.shape[0]
    return pl.pallas_call(
        _matmul_tile,
        grid=(n_tokens // _TM,),
        in_specs=[
            pl.BlockSpec((_TM, _HIDDEN), lambda i: (i, 0)),
            pl.BlockSpec((_EXPERTS, _HIDDEN), lambda i: (0, 0)),
        ],
        out_specs=pl.BlockSpec((_EXPERTS, _TM), lambda i: (0, i)),
        out_shape=jax.ShapeDtypeStruct((_EXPERTS, n_tokens), jnp.float32),
        compiler_params=pltpu.CompilerParams(
            dimension_semantics=("parallel",)),
    )(x, W)


def _make_sc_route(n_tokens):
    tw = n_tokens // _NW
    mesh = plsc.VectorSubcoreMesh(core_axis_name="c", subcore_axis_name="s")
    neg = jnp.finfo(jnp.float32).min

    @functools.partial(
        pl.kernel, mesh=mesh,
        out_type=[
            jax.ShapeDtypeStruct((2, n_tokens), jnp.float32),
            jax.ShapeDtypeStruct((2, n_tokens), jnp.int32),
        ],
        scratch_types=[
            pltpu.VMEM((_EXPERTS, tw), jnp.float32),
            pltpu.VMEM((2, tw), jnp.float32),
            pltpu.VMEM((2, tw), jnp.int32),
        ],
    )
    def route(lt_hbm, sc_hbm, ix_hbm, buf, sc_v, ix_v):
        wid = lax.axis_index("s") * _NC + lax.axis_index("c")
        base = wid * tw
        pltpu.sync_copy(lt_hbm.at[:, pl.ds(base, tw)], buf)

        def group(g, carry):
            off = g * _L
            m1 = jnp.full((_L,), neg, jnp.float32)
            m2 = jnp.full((_L,), neg, jnp.float32)
            i1 = jnp.zeros((_L,), jnp.int32)
            i2 = jnp.zeros((_L,), jnp.int32)
            for e in range(_EXPERTS):
                v = buf[e, pl.ds(off, _L)]
                e_vec = jnp.full((_L,), e, jnp.int32)
                ge1 = v > m1
                cand = jnp.where(ge1, m1, v)
                cand_i = jnp.where(ge1, i1, e_vec)
                m1 = jnp.where(ge1, v, m1)
                i1 = jnp.where(ge1, e_vec, i1)
                ge2 = cand > m2
                m2 = jnp.where(ge2, cand, m2)
                i2 = jnp.where(ge2, cand_i, i2)
            e2 = jnp.exp(m2 - m1)
            s1 = 1.0 / (1.0 + e2)
            sc_v[0, pl.ds(off, _L)] = s1
            sc_v[1, pl.ds(off, _L)] = e2 * s1
            ix_v[0, pl.ds(off, _L)] = i1
            ix_v[1, pl.ds(off, _L)] = i2
            return carry

        lax.fori_loop(0, tw // _L, group, 0)
        pltpu.sync_copy(sc_v, sc_hbm.at[:, pl.ds(base, tw)])
        pltpu.sync_copy(ix_v, ix_hbm.at[:, pl.ds(base, tw)])

    return route


def kernel(x, W):
    n_tokens = x.shape[0]
    logits_t = _tc_logits_t(x, W)
    scores_t, idx_t = _make_sc_route(n_tokens)(logits_t)
    return (scores_t.T, idx_t.T)


# speedup vs baseline: 1.0239x; 1.0239x over previous
import jax
import jax.numpy as jnp
from jax.experimental import pallas as pl
from jax.experimental.pallas import tpu as pltpu

_HIDDEN = 4096
_EXPERTS = 64
_TM = 1024


def _gate_tile(x_ref, w_ref, scores_ref, idx_ref):
    x = x_ref[...]
    w = w_ref[...]
    logits = jax.lax.dot_general(
        x, w, (((1,), (1,)), ((), ())),
        preferred_element_type=jnp.float32)

    cols = jax.lax.broadcasted_iota(jnp.int32, logits.shape, 1)
    big = jnp.int32(_EXPERTS)

    m1 = jnp.max(logits, axis=1, keepdims=True)
    i1 = jnp.min(jnp.where(logits == m1, cols, big), axis=1, keepdims=True)
    masked = jnp.where(cols == i1, jnp.finfo(jnp.float32).min, logits)
    m2 = jnp.max(masked, axis=1, keepdims=True)
    i2 = jnp.min(jnp.where(masked == m2, cols, big), axis=1, keepdims=True)

    e2 = jnp.exp(m2 - m1)
    s1 = 1.0 / (1.0 + e2)
    scores_ref[...] = jnp.concatenate([s1, e2 * s1], axis=1)
    idx_ref[...] = jnp.concatenate([i1, i2], axis=1)


def kernel(x, W):
    n_tokens = x.shape[0]
    grid = (n_tokens // _TM,)
    scores, idx = pl.pallas_call(
        _gate_tile,
        grid=grid,
        in_specs=[
            pl.BlockSpec((_TM, _HIDDEN), lambda i: (i, 0)),
            pl.BlockSpec((_EXPERTS, _HIDDEN), lambda i: (0, 0)),
        ],
        out_specs=[
            pl.BlockSpec((_TM, 2), lambda i: (i, 0)),
            pl.BlockSpec((_TM, 2), lambda i: (i, 0)),
        ],
        out_shape=[
            jax.ShapeDtypeStruct((n_tokens, 2), jnp.float32),
            jax.ShapeDtypeStruct((n_tokens, 2), jnp.int32),
        ],
        compiler_params=pltpu.CompilerParams(
            dimension_semantics=("parallel",)),
    )(x, W)
    return (scores, idx)
